# baseline (speedup 1.0000x reference)
import functools
import jax
import jax.numpy as jnp
from jax import lax
from jax.experimental import pallas as pl
from jax.experimental.pallas import tpu as pltpu
from jax.experimental.pallas import tpu_sc as plsc

N = 10000
E = 320000
H = 128
R = 32
G = 64
NVOC = 100

NC = 2
NS = 16
NW = NC * NS
CH = 80
SUP = 400
NSUB = SUP // CH
NPAD = 10240
RPT = NPAD // NS

EBLOCKS = [(0, 51200, 80), (51200, 179200, 56), (230400, 89600, 56)]

_mesh = plsc.VectorSubcoreMesh(core_axis_name="c", subcore_axis_name="s")


def _make_gather(e0, cnt):
    spw = cnt // NS
    nsup = spw // SUP
    assert spw % SUP == 0 and nsup % 2 == 0

    @functools.partial(
        pl.kernel,
        mesh=_mesh,
        out_type=[
            jax.ShapeDtypeStruct((cnt, H), jnp.float32),
            jax.ShapeDtypeStruct((cnt, H), jnp.float32),
        ],
        scratch_types=[
            pltpu.VMEM((spw,), jnp.int32),
            pltpu.VMEM((SUP, H), jnp.float32),
            pltpu.VMEM((SUP, H), jnp.float32),
            pltpu.SemaphoreType.DMA,
            pltpu.SemaphoreType.DMA,
            pltpu.SemaphoreType.DMA,
            pltpu.SemaphoreType.DMA,
        ],
    )
    def _gather(h_hbm, src_hbm, dst_hbm, zs_hbm, zd_hbm, idx, bufa, bufb, gsa, gsb, wsa, wsb):
        c = lax.axis_index("c")
        s = lax.axis_index("s")
        base = s * spw

        @pl.when(c == 0)
        def _():
            pltpu.sync_copy(src_hbm.at[pl.ds(e0 + base, spw)], idx)

        @pl.when(c == 1)
        def _():
            pltpu.sync_copy(dst_hbm.at[pl.ds(e0 + base, spw)], idx)

        def issue_gathers(t, buf, gsem):
            for k in range(NSUB):
                pltpu.async_copy(
                    h_hbm.at[idx.at[pl.ds(t * SUP + k * CH, CH)]],
                    buf.at[pl.ds(k * CH, CH)], gsem)

        def wait_gathers(buf, gsem):
            pltpu.make_async_copy(h_hbm.at[pl.ds(0, SUP)], buf, gsem).wait()

        def issue_write(t, buf, wsem):
            @pl.when(c == 0)
            def _():
                pltpu.async_copy(buf, zs_hbm.at[pl.ds(base + t * SUP, SUP)], wsem)

            @pl.when(c == 1)
            def _():
                pltpu.async_copy(buf, zd_hbm.at[pl.ds(base + t * SUP, SUP)], wsem)

        def wait_write(buf, wsem):
            pltpu.make_async_copy(buf, zs_hbm.at[pl.ds(base, SUP)], wsem).wait()

        issue_gathers(0, bufa, gsa)

        def phase(t, buf, gsem, wsem, obuf, ogsem, owsem):
            @pl.when(t >= 1)
            def _():
                wait_write(obuf, owsem)

            @pl.when(t + 1 < nsup)
            def _():
                issue_gathers(t + 1, obuf, ogsem)

            wait_gathers(buf, gsem)
            issue_write(t, buf, wsem)

        def body(o, carry):
            phase(2 * o, bufa, gsa, wsa, bufb, gsb, wsb)
            phase(2 * o + 1, bufb, gsb, wsb, bufa, gsa, wsa)
            return carry

        lax.fori_loop(0, nsup // 2, body, 0)
        wait_write(bufb, wsb)

    return _gather


def _make_scatter(cnt, ch, msup):
    epw = cnt // NW
    msub = msup // ch
    nmsup = epw // msup
    nch = epw // ch
    assert epw % msup == 0 and msup % ch == 0 and nmsup % 2 == 0

    @functools.partial(
        pl.kernel,
        mesh=_mesh,
        out_type=[
            jax.ShapeDtypeStruct((NPAD, H), jnp.float32),
            jax.ShapeDtypeStruct((NPAD, H), jnp.float32),
        ],
        scratch_types=[
            pltpu.VMEM((nch, ch), jnp.int32),
            pltpu.VMEM((msup, H), jnp.float32),
            pltpu.VMEM((msup, H), jnp.float32),
            pltpu.VMEM_SHARED((NPAD, H), jnp.float32),
            pltpu.SemaphoreType.DMA,
            pltpu.SemaphoreType.DMA,
        ],
    )
    def _scatter(m_hbm, dst3_hbm, zero_hbm, agg0_hbm, agg1_hbm, di, bufa, bufb, acc, msa, msb):
        c = lax.axis_index("c")
        s = lax.axis_index("s")
        wid = c * NS + s
        base = wid * epw
        r0 = s * RPT

        pltpu.sync_copy(zero_hbm.at[pl.ds(r0, RPT)], acc.at[pl.ds(r0, RPT)])
        pltpu.sync_copy(dst3_hbm.at[wid], di)
        plsc.subcore_barrier()

        def issue_load(t, buf, sem):
            pltpu.async_copy(m_hbm.at[pl.ds(base + t * msup, msup)], buf, sem)

        def wait_load(buf, sem):
            pltpu.make_async_copy(m_hbm.at[pl.ds(0, msup)], buf, sem).wait()

        def phase(t, buf, sem, obuf, osem):
            @pl.when(t + 1 < nmsup)
            def _():
                issue_load(t + 1, obuf, osem)

            wait_load(buf, sem)
            for k in range(msub):
                pltpu.sync_copy(buf.at[pl.ds(k * ch, ch)],
                                acc.at[di.at[t * msub + k]], add=True)

        issue_load(0, bufa, msa)

        def body(o, carry):
            phase(2 * o, bufa, msa, bufb, msb)
            phase(2 * o + 1, bufb, msb, bufa, msa)
            return carry

        lax.fori_loop(0, nmsup // 2, body, 0)

        plsc.subcore_barrier()

        @pl.when(c == 0)
        def _():
            pltpu.sync_copy(acc.at[pl.ds(r0, RPT)], agg0_hbm.at[pl.ds(r0, RPT)])

        @pl.when(c == 1)
        def _():
            pltpu.sync_copy(acc.at[pl.ds(r0, RPT)], agg1_hbm.at[pl.ds(r0, RPT)])

    return _scatter


_gathers = [_make_gather(e0, cnt) for e0, cnt, _ in EBLOCKS]
_scatters = [_make_scatter(cnt, ch, ch) for _, cnt, ch in EBLOCKS]


BN = 2000
GN = N // BN
BE = 2560


def _emb_body(x_ref, emb_ref, h_ref):
    xb = x_ref[0, 0, :]
    onehot = (xb[:, None] == lax.broadcasted_iota(jnp.int32, (BN, NVOC), 1))
    h_ref[...] = jnp.dot(onehot.astype(jnp.float32), emb_ref[...],
                         preferred_element_type=jnp.float32)


_emb_call = pl.pallas_call(
    _emb_body,
    grid=(GN,),
    in_specs=[
        pl.BlockSpec((1, 1, BN), lambda i: (i, 0, 0)),
        pl.BlockSpec((NVOC, H), lambda i: (0, 0)),
    ],
    out_specs=pl.BlockSpec((BN, H), lambda i: (i, 0)),
    out_shape=jax.ShapeDtypeStruct((N, H), jnp.float32),
)


def _edge_body(zd_ref, zs_ref, ea_ref, wf_ref, bf_ref, ws_ref, bs_ref, m_ref):
    zd = zd_ref[...]
    zs = zs_ref[...]
    ea_t = ea_ref[...]

    def gate(w_ref, b_ref):
        return (jnp.dot(zd, w_ref[0:H, :], preferred_element_type=jnp.float32)
                + jnp.dot(zs, w_ref[H:2 * H, :], preferred_element_type=jnp.float32)
                + lax.dot_general(ea_t, w_ref[2 * H:, :], (((0,), (0,)), ((), ())),
                                  preferred_element_type=jnp.float32)
                + b_ref[...])

    f = gate(wf_ref, bf_ref)
    s = gate(ws_ref, bs_ref)
    sig = 1.0 / (1.0 + jnp.exp(-f))
    sp = jnp.maximum(s, 0.0) + jnp.log(1.0 + jnp.exp(-jnp.abs(s)))
    m_ref[...] = sig * sp


def _make_edge(cnt, e0):
    blk0 = e0 // BE
    return pl.pallas_call(
        _edge_body,
        grid=(cnt // BE,),
        in_specs=[
            pl.BlockSpec((BE, H), lambda i: (i, 0)),
            pl.BlockSpec((BE, H), lambda i: (i, 0)),
            pl.BlockSpec((R, BE), lambda i: (0, blk0 + i)),
            pl.BlockSpec((2 * H + R, H), lambda i: (0, 0)),
            pl.BlockSpec((H,), lambda i: (0,)),
            pl.BlockSpec((2 * H + R, H), lambda i: (0, 0)),
            pl.BlockSpec((H,), lambda i: (0,)),
        ],
        out_specs=pl.BlockSpec((BE, H), lambda i: (i, 0)),
        out_shape=jax.ShapeDtypeStruct((cnt, H), jnp.float32),
    )


_edges = [_make_edge(cnt, e0) for e0, cnt, _ in EBLOCKS]


NAGG = 2 * len(EBLOCKS)


def _agg_sum(arefs):
    t = arefs[0][...]
    for a in arefs[1:]:
        t = t + a[...]
    return t


def _combine_body(h_ref, *rest):
    arefs, o_ref = rest[:NAGG], rest[NAGG]
    o_ref[...] = jnp.maximum(h_ref[...] + _agg_sum(arefs), 0.0)


_combine_call = pl.pallas_call(
    _combine_body,
    grid=(GN,),
    in_specs=[pl.BlockSpec((BN, H), lambda i: (i, 0)) for _ in range(1 + NAGG)],
    out_specs=pl.BlockSpec((BN, H), lambda i: (i, 0)),
    out_shape=jax.ShapeDtypeStruct((N, H), jnp.float32),
)


def _pool_body(h_ref, *rest):
    arefs = rest[:NAGG]
    b_ref, wl_ref, bl_ref, o_ref, sums, cnts = rest[NAGG:]
    i = pl.program_id(0)

    @pl.when(i == 0)
    def _():
        sums[...] = jnp.zeros_like(sums)
        cnts[...] = jnp.zeros_like(cnts)

    h3 = jnp.maximum(h_ref[...] + _agg_sum(arefs), 0.0)
    bb = b_ref[0, 0, :]
    onehot = (bb[:, None] == lax.broadcasted_iota(jnp.int32, (BN, G), 1)).astype(jnp.float32)
    sums[...] += lax.dot_general(onehot, h3, (((0,), (0,)), ((), ())),
                                 preferred_element_type=jnp.float32)
    cnts[...] += jnp.broadcast_to(jnp.sum(onehot, axis=0)[:, None], (G, H))

    @pl.when(i == GN - 1)
    def _():
        pooled = sums[...] / jnp.maximum(cnts[...], 1.0)
        o_ref[...] = jnp.dot(pooled, wl_ref[...],
                             preferred_element_type=jnp.float32) + bl_ref[...]


_pool_call = pl.pallas_call(
    _pool_body,
    grid=(GN,),
    in_specs=[pl.BlockSpec((BN, H), lambda i: (i, 0)) for _ in range(1 + NAGG)] + [
        pl.BlockSpec((1, 1, BN), lambda i: (i, 0, 0)),
        pl.BlockSpec((H, H), lambda i: (0, 0)),
        pl.BlockSpec((H,), lambda i: (0,)),
    ],
    out_specs=pl.BlockSpec((G, H), lambda i: (0, 0)),
    out_shape=jax.ShapeDtypeStruct((G, H), jnp.float32),
    scratch_shapes=[
        pltpu.VMEM((G, H), jnp.float32),
        pltpu.VMEM((G, H), jnp.float32),
    ],
)


def kernel(x, edge_index, edge_attr, batch, emb,
           Wf1, bf1, Ws1, bs1, Wf2, bf2, Ws2, bs2, Wf3, bf3, Ws3, bs3, Wl, bl):
    src = edge_index[0].astype(jnp.int32)
    dst = edge_index[1].astype(jnp.int32)
    eat = edge_attr.T
    dst3s = [dst[e0:e0 + cnt].reshape(NW, (cnt // NW) // ch, ch)
             for e0, cnt, ch in EBLOCKS]
    x3 = x.reshape(GN, 1, BN).astype(jnp.int32)
    b3 = batch.reshape(GN, 1, BN).astype(jnp.int32)
    zero = jnp.zeros((NPAD, H), jnp.float32)

    h = _emb_call(x3, emb)
    layers = [(Wf1, bf1, Ws1, bs1), (Wf2, bf2, Ws2, bs2), (Wf3, bf3, Ws3, bs3)]
    agg = None
    for li, (Wf, bf, Ws, bs) in enumerate(layers):
        if li > 0:
            h = _combine_call(h, *agg)
        zs = [g(h, src, dst) for g in _gathers]
        ms = [e(zsd[1], zsd[0], eat, Wf, bf, Ws, bs)
              for e, zsd in zip(_edges, zs)]
        agg = [a for s, m, d3 in zip(_scatters, ms, dst3s)
               for a in s(m, d3, zero)]

    return _pool_call(h, *agg, b3, Wl, bl)


# speedup vs baseline: 1.0458x; 1.0458x over previous
import functools
import jax
import jax.numpy as jnp
from jax import lax
from jax.experimental import pallas as pl
from jax.experimental.pallas import tpu as pltpu
from jax.experimental.pallas import tpu_sc as plsc

N = 10000
E = 320000
H = 128
R = 32
G = 64
NVOC = 100

NC = 2
NS = 16
NW = NC * NS
CH = 80
SUP = 400
NSUB = SUP // CH
NPAD = 10240
RPT = NPAD // NS

EBLOCKS = [(0, 89600, 56), (89600, 140800, 88), (230400, 89600, 56)]

_mesh = plsc.VectorSubcoreMesh(core_axis_name="c", subcore_axis_name="s")


def _make_gather(e0, cnt):
    spw = cnt // NS
    nsup = spw // SUP
    assert spw % SUP == 0 and nsup % 2 == 0

    @functools.partial(
        pl.kernel,
        mesh=_mesh,
        out_type=[
            jax.ShapeDtypeStruct((cnt, H), jnp.float32),
            jax.ShapeDtypeStruct((cnt, H), jnp.float32),
        ],
        scratch_types=[
            pltpu.VMEM((spw,), jnp.int32),
            pltpu.VMEM((SUP, H), jnp.float32),
            pltpu.VMEM((SUP, H), jnp.float32),
            pltpu.SemaphoreType.DMA,
            pltpu.SemaphoreType.DMA,
            pltpu.SemaphoreType.DMA,
            pltpu.SemaphoreType.DMA,
        ],
    )
    def _gather(h_hbm, src_hbm, dst_hbm, zs_hbm, zd_hbm, idx, bufa, bufb, gsa, gsb, wsa, wsb):
        c = lax.axis_index("c")
        s = lax.axis_index("s")
        base = s * spw

        @pl.when(c == 0)
        def _():
            pltpu.sync_copy(src_hbm.at[pl.ds(e0 + base, spw)], idx)

        @pl.when(c == 1)
        def _():
            pltpu.sync_copy(dst_hbm.at[pl.ds(e0 + base, spw)], idx)

        def issue_gathers(t, buf, gsem):
            for k in range(NSUB):
                pltpu.async_copy(
                    h_hbm.at[idx.at[pl.ds(t * SUP + k * CH, CH)]],
                    buf.at[pl.ds(k * CH, CH)], gsem)

        def wait_gathers(buf, gsem):
            pltpu.make_async_copy(h_hbm.at[pl.ds(0, SUP)], buf, gsem).wait()

        def issue_write(t, buf, wsem):
            @pl.when(c == 0)
            def _():
                pltpu.async_copy(buf, zs_hbm.at[pl.ds(base + t * SUP, SUP)], wsem)

            @pl.when(c == 1)
            def _():
                pltpu.async_copy(buf, zd_hbm.at[pl.ds(base + t * SUP, SUP)], wsem)

        def wait_write(buf, wsem):
            pltpu.make_async_copy(buf, zs_hbm.at[pl.ds(base, SUP)], wsem).wait()

        issue_gathers(0, bufa, gsa)

        def phase(t, buf, gsem, wsem, obuf, ogsem, owsem):
            @pl.when(t >= 1)
            def _():
                wait_write(obuf, owsem)

            @pl.when(t + 1 < nsup)
            def _():
                issue_gathers(t + 1, obuf, ogsem)

            wait_gathers(buf, gsem)
            issue_write(t, buf, wsem)

        def body(o, carry):
            phase(2 * o, bufa, gsa, wsa, bufb, gsb, wsb)
            phase(2 * o + 1, bufb, gsb, wsb, bufa, gsa, wsa)
            return carry

        lax.fori_loop(0, nsup // 2, body, 0)
        wait_write(bufb, wsb)

    return _gather


def _make_scatter(cnt, ch, msup):
    epw = cnt // NW
    msub = msup // ch
    nmsup = epw // msup
    nch = epw // ch
    assert epw % msup == 0 and msup % ch == 0 and nmsup % 2 == 0

    @functools.partial(
        pl.kernel,
        mesh=_mesh,
        out_type=[
            jax.ShapeDtypeStruct((NPAD, H), jnp.float32),
            jax.ShapeDtypeStruct((NPAD, H), jnp.float32),
        ],
        scratch_types=[
            pltpu.VMEM((nch, ch), jnp.int32),
            pltpu.VMEM((msup, H), jnp.float32),
            pltpu.VMEM((msup, H), jnp.float32),
            pltpu.VMEM_SHARED((NPAD, H), jnp.float32),
            pltpu.SemaphoreType.DMA,
            pltpu.SemaphoreType.DMA,
        ],
    )
    def _scatter(m_hbm, dst3_hbm, zero_hbm, agg0_hbm, agg1_hbm, di, bufa, bufb, acc, msa, msb):
        c = lax.axis_index("c")
        s = lax.axis_index("s")
        wid = c * NS + s
        base = wid * epw
        r0 = s * RPT

        pltpu.sync_copy(zero_hbm.at[pl.ds(r0, RPT)], acc.at[pl.ds(r0, RPT)])
        pltpu.sync_copy(dst3_hbm.at[wid], di)
        plsc.subcore_barrier()

        def issue_load(t, buf, sem):
            pltpu.async_copy(m_hbm.at[pl.ds(base + t * msup, msup)], buf, sem)

        def wait_load(buf, sem):
            pltpu.make_async_copy(m_hbm.at[pl.ds(0, msup)], buf, sem).wait()

        def phase(t, buf, sem, obuf, osem):
            @pl.when(t + 1 < nmsup)
            def _():
                issue_load(t + 1, obuf, osem)

            wait_load(buf, sem)
            for k in range(msub):
                pltpu.sync_copy(buf.at[pl.ds(k * ch, ch)],
                                acc.at[di.at[t * msub + k]], add=True)

        issue_load(0, bufa, msa)

        def body(o, carry):
            phase(2 * o, bufa, msa, bufb, msb)
            phase(2 * o + 1, bufb, msb, bufa, msa)
            return carry

        lax.fori_loop(0, nmsup // 2, body, 0)

        plsc.subcore_barrier()

        @pl.when(c == 0)
        def _():
            pltpu.sync_copy(acc.at[pl.ds(r0, RPT)], agg0_hbm.at[pl.ds(r0, RPT)])

        @pl.when(c == 1)
        def _():
            pltpu.sync_copy(acc.at[pl.ds(r0, RPT)], agg1_hbm.at[pl.ds(r0, RPT)])

    return _scatter


_gathers = [_make_gather(e0, cnt) for e0, cnt, _ in EBLOCKS]
_scatters = [_make_scatter(cnt, ch, ch) for _, cnt, ch in EBLOCKS]


BN = 2000
GN = N // BN
BE = 6400


def _emb_body(x_ref, emb_ref, h_ref):
    xb = x_ref[0, 0, :]
    onehot = (xb[:, None] == lax.broadcasted_iota(jnp.int32, (BN, NVOC), 1))
    h_ref[...] = jnp.dot(onehot.astype(jnp.float32), emb_ref[...],
                         preferred_element_type=jnp.float32)


_emb_call = pl.pallas_call(
    _emb_body,
    grid=(GN,),
    in_specs=[
        pl.BlockSpec((1, 1, BN), lambda i: (i, 0, 0)),
        pl.BlockSpec((NVOC, H), lambda i: (0, 0)),
    ],
    out_specs=pl.BlockSpec((BN, H), lambda i: (i, 0)),
    out_shape=jax.ShapeDtypeStruct((N, H), jnp.float32),
)


def _edge_body(zd_ref, zs_ref, ea_ref, wf_ref, bf_ref, ws_ref, bs_ref, m_ref):
    zd = zd_ref[...]
    zs = zs_ref[...]
    ea_t = ea_ref[...]

    def gate(w_ref, b_ref):
        return (jnp.dot(zd, w_ref[0:H, :], preferred_element_type=jnp.float32)
                + jnp.dot(zs, w_ref[H:2 * H, :], preferred_element_type=jnp.float32)
                + lax.dot_general(ea_t, w_ref[2 * H:, :], (((0,), (0,)), ((), ())),
                                  preferred_element_type=jnp.float32)
                + b_ref[...])

    f = gate(wf_ref, bf_ref)
    s = gate(ws_ref, bs_ref)
    sig = 1.0 / (1.0 + jnp.exp(-f))
    sp = jnp.maximum(s, 0.0) + jnp.log(1.0 + jnp.exp(-jnp.abs(s)))
    m_ref[...] = sig * sp


def _make_edge(cnt, e0):
    blk0 = e0 // BE
    return pl.pallas_call(
        _edge_body,
        grid=(cnt // BE,),
        in_specs=[
            pl.BlockSpec((BE, H), lambda i: (i, 0)),
            pl.BlockSpec((BE, H), lambda i: (i, 0)),
            pl.BlockSpec((R, BE), lambda i: (0, blk0 + i)),
            pl.BlockSpec((2 * H + R, H), lambda i: (0, 0)),
            pl.BlockSpec((H,), lambda i: (0,)),
            pl.BlockSpec((2 * H + R, H), lambda i: (0, 0)),
            pl.BlockSpec((H,), lambda i: (0,)),
        ],
        out_specs=pl.BlockSpec((BE, H), lambda i: (i, 0)),
        out_shape=jax.ShapeDtypeStruct((cnt, H), jnp.float32),
    )


_edges = [_make_edge(cnt, e0) for e0, cnt, _ in EBLOCKS]


NAGG = 2 * len(EBLOCKS)


def _agg_sum(arefs):
    t = arefs[0][...]
    for a in arefs[1:]:
        t = t + a[...]
    return t


def _combine_body(h_ref, *rest):
    arefs, o_ref = rest[:NAGG], rest[NAGG]
    o_ref[...] = jnp.maximum(h_ref[...] + _agg_sum(arefs), 0.0)


_combine_call = pl.pallas_call(
    _combine_body,
    grid=(GN,),
    in_specs=[pl.BlockSpec((BN, H), lambda i: (i, 0)) for _ in range(1 + NAGG)],
    out_specs=pl.BlockSpec((BN, H), lambda i: (i, 0)),
    out_shape=jax.ShapeDtypeStruct((N, H), jnp.float32),
)


def _pool_body(h_ref, *rest):
    arefs = rest[:NAGG]
    b_ref, wl_ref, bl_ref, o_ref, sums, cnts = rest[NAGG:]
    i = pl.program_id(0)

    @pl.when(i == 0)
    def _():
        sums[...] = jnp.zeros_like(sums)
        cnts[...] = jnp.zeros_like(cnts)

    h3 = jnp.maximum(h_ref[...] + _agg_sum(arefs), 0.0)
    bb = b_ref[0, 0, :]
    onehot = (bb[:, None] == lax.broadcasted_iota(jnp.int32, (BN, G), 1)).astype(jnp.float32)
    sums[...] += lax.dot_general(onehot, h3, (((0,), (0,)), ((), ())),
                                 preferred_element_type=jnp.float32)
    cnts[...] += jnp.broadcast_to(jnp.sum(onehot, axis=0)[:, None], (G, H))

    @pl.when(i == GN - 1)
    def _():
        pooled = sums[...] / jnp.maximum(cnts[...], 1.0)
        o_ref[...] = jnp.dot(pooled, wl_ref[...],
                             preferred_element_type=jnp.float32) + bl_ref[...]


_pool_call = pl.pallas_call(
    _pool_body,
    grid=(GN,),
    in_specs=[pl.BlockSpec((BN, H), lambda i: (i, 0)) for _ in range(1 + NAGG)] + [
        pl.BlockSpec((1, 1, BN), lambda i: (i, 0, 0)),
        pl.BlockSpec((H, H), lambda i: (0, 0)),
        pl.BlockSpec((H,), lambda i: (0,)),
    ],
    out_specs=pl.BlockSpec((G, H), lambda i: (0, 0)),
    out_shape=jax.ShapeDtypeStruct((G, H), jnp.float32),
    scratch_shapes=[
        pltpu.VMEM((G, H), jnp.float32),
        pltpu.VMEM((G, H), jnp.float32),
    ],
)


def kernel(x, edge_index, edge_attr, batch, emb,
           Wf1, bf1, Ws1, bs1, Wf2, bf2, Ws2, bs2, Wf3, bf3, Ws3, bs3, Wl, bl):
    src = edge_index[0].astype(jnp.int32)
    dst = edge_index[1].astype(jnp.int32)
    eat = edge_attr.T
    dst3s = [dst[e0:e0 + cnt].reshape(NW, (cnt // NW) // ch, ch)
             for e0, cnt, ch in EBLOCKS]
    x3 = x.reshape(GN, 1, BN).astype(jnp.int32)
    b3 = batch.reshape(GN, 1, BN).astype(jnp.int32)
    zero = jnp.zeros((NPAD, H), jnp.float32)

    h = _emb_call(x3, emb)
    layers = [(Wf1, bf1, Ws1, bs1), (Wf2, bf2, Ws2, bs2), (Wf3, bf3, Ws3, bs3)]
    agg = None
    for li, (Wf, bf, Ws, bs) in enumerate(layers):
        if li > 0:
            h = _combine_call(h, *agg)
        zs = [g(h, src, dst) for g in _gathers]
        ms = [e(zsd[1], zsd[0], eat, Wf, bf, Ws, bs)
              for e, zsd in zip(_edges, zs)]
        agg = [a for s, m, d3 in zip(_scatters, ms, dst3s)
               for a in s(m, d3, zero)]

    return _pool_call(h, *agg, b3, Wl, bl)


# speedup vs baseline: 1.0514x; 1.0054x over previous
import functools
import jax
import jax.numpy as jnp
from jax import lax
from jax.experimental import pallas as pl
from jax.experimental.pallas import tpu as pltpu
from jax.experimental.pallas import tpu_sc as plsc

N = 10000
E = 320000
H = 128
R = 32
G = 64
NVOC = 100

NC = 2
NS = 16
NW = NC * NS
CH = 80
SUP = 400
NSUB = SUP // CH
NPAD = 10240
RPT = NPAD // NS

EBLOCKS = [(0, 89600, 56), (89600, 140800, 88), (230400, 89600, 56)]

_mesh = plsc.VectorSubcoreMesh(core_axis_name="c", subcore_axis_name="s")


def _make_gather(e0, cnt):
    spw = cnt // NS
    nsup = spw // SUP
    assert spw % SUP == 0 and nsup % 2 == 0

    @functools.partial(
        pl.kernel,
        mesh=_mesh,
        out_type=[
            jax.ShapeDtypeStruct((cnt, H), jnp.float32),
            jax.ShapeDtypeStruct((cnt, H), jnp.float32),
        ],
        scratch_types=[
            pltpu.VMEM((spw,), jnp.int32),
            pltpu.VMEM((SUP, H), jnp.float32),
            pltpu.VMEM((SUP, H), jnp.float32),
            pltpu.SemaphoreType.DMA,
            pltpu.SemaphoreType.DMA,
            pltpu.SemaphoreType.DMA,
            pltpu.SemaphoreType.DMA,
        ],
    )
    def _gather(h_hbm, src_hbm, dst_hbm, zs_hbm, zd_hbm, idx, bufa, bufb, gsa, gsb, wsa, wsb):
        c = lax.axis_index("c")
        s = lax.axis_index("s")
        base = s * spw

        @pl.when(c == 0)
        def _():
            pltpu.sync_copy(src_hbm.at[pl.ds(e0 + base, spw)], idx)

        @pl.when(c == 1)
        def _():
            pltpu.sync_copy(dst_hbm.at[pl.ds(e0 + base, spw)], idx)

        def issue_gathers(t, buf, gsem):
            for k in range(NSUB):
                pltpu.async_copy(
                    h_hbm.at[idx.at[pl.ds(t * SUP + k * CH, CH)]],
                    buf.at[pl.ds(k * CH, CH)], gsem)

        def wait_gathers(buf, gsem):
            pltpu.make_async_copy(h_hbm.at[pl.ds(0, SUP)], buf, gsem).wait()

        def issue_write(t, buf, wsem):
            @pl.when(c == 0)
            def _():
                pltpu.async_copy(buf, zs_hbm.at[pl.ds(base + t * SUP, SUP)], wsem)

            @pl.when(c == 1)
            def _():
                pltpu.async_copy(buf, zd_hbm.at[pl.ds(base + t * SUP, SUP)], wsem)

        def wait_write(buf, wsem):
            pltpu.make_async_copy(buf, zs_hbm.at[pl.ds(base, SUP)], wsem).wait()

        issue_gathers(0, bufa, gsa)

        def phase(t, buf, gsem, wsem, obuf, ogsem, owsem):
            @pl.when(t >= 1)
            def _():
                wait_write(obuf, owsem)

            @pl.when(t + 1 < nsup)
            def _():
                issue_gathers(t + 1, obuf, ogsem)

            wait_gathers(buf, gsem)
            issue_write(t, buf, wsem)

        def body(o, carry):
            phase(2 * o, bufa, gsa, wsa, bufb, gsb, wsb)
            phase(2 * o + 1, bufb, gsb, wsb, bufa, gsa, wsa)
            return carry

        lax.fori_loop(0, nsup // 2, body, 0)
        wait_write(bufb, wsb)

    return _gather


def _make_scatter(cnt, ch, msup):
    epw = cnt // NW
    msub = msup // ch
    nmsup = epw // msup
    nch = epw // ch
    assert epw % msup == 0 and msup % ch == 0 and nmsup % 2 == 0

    @functools.partial(
        pl.kernel,
        mesh=_mesh,
        out_type=[
            jax.ShapeDtypeStruct((NPAD, H), jnp.float32),
            jax.ShapeDtypeStruct((NPAD, H), jnp.float32),
        ],
        scratch_types=[
            pltpu.VMEM((nch, ch), jnp.int32),
            pltpu.VMEM((msup, H), jnp.float32),
            pltpu.VMEM((msup, H), jnp.float32),
            pltpu.VMEM_SHARED((NPAD, H), jnp.float32),
            pltpu.SemaphoreType.DMA,
            pltpu.SemaphoreType.DMA,
        ],
    )
    def _scatter(m_hbm, dst3_hbm, zero_hbm, agg0_hbm, agg1_hbm, di, bufa, bufb, acc, msa, msb):
        c = lax.axis_index("c")
        s = lax.axis_index("s")
        wid = c * NS + s
        base = wid * epw
        r0 = s * RPT

        pltpu.sync_copy(zero_hbm.at[pl.ds(r0, RPT)], acc.at[pl.ds(r0, RPT)])
        pltpu.sync_copy(dst3_hbm.at[wid], di)
        plsc.subcore_barrier()

        def issue_load(t, buf, sem):
            pltpu.async_copy(m_hbm.at[pl.ds(base + t * msup, msup)], buf, sem)

        def wait_load(buf, sem):
            pltpu.make_async_copy(m_hbm.at[pl.ds(0, msup)], buf, sem).wait()

        def phase(t, buf, sem, obuf, osem):
            @pl.when(t + 1 < nmsup)
            def _():
                issue_load(t + 1, obuf, osem)

            wait_load(buf, sem)
            for k in range(msub):
                pltpu.sync_copy(buf.at[pl.ds(k * ch, ch)],
                                acc.at[di.at[t * msub + k]], add=True)

        issue_load(0, bufa, msa)

        def body(o, carry):
            phase(2 * o, bufa, msa, bufb, msb)
            phase(2 * o + 1, bufb, msb, bufa, msa)
            return carry

        lax.fori_loop(0, nmsup // 2, body, 0)

        plsc.subcore_barrier()

        @pl.when(c == 0)
        def _():
            pltpu.sync_copy(acc.at[pl.ds(r0, RPT)], agg0_hbm.at[pl.ds(r0, RPT)])

        @pl.when(c == 1)
        def _():
            pltpu.sync_copy(acc.at[pl.ds(r0, RPT)], agg1_hbm.at[pl.ds(r0, RPT)])

    return _scatter


_gathers = [_make_gather(e0, cnt) for e0, cnt, _ in EBLOCKS]
_scatters = [_make_scatter(cnt, ch, ch) for _, cnt, ch in EBLOCKS]


BN = 2000
GN = N // BN
BE = 2560


def _emb_body(x_ref, emb_ref, h_ref):
    xb = x_ref[0, 0, :]
    onehot = (xb[:, None] == lax.broadcasted_iota(jnp.int32, (BN, NVOC), 1))
    h_ref[...] = jnp.dot(onehot.astype(jnp.float32), emb_ref[...],
                         preferred_element_type=jnp.float32)


_emb_call = pl.pallas_call(
    _emb_body,
    grid=(GN,),
    in_specs=[
        pl.BlockSpec((1, 1, BN), lambda i: (i, 0, 0)),
        pl.BlockSpec((NVOC, H), lambda i: (0, 0)),
    ],
    out_specs=pl.BlockSpec((BN, H), lambda i: (i, 0)),
    out_shape=jax.ShapeDtypeStruct((N, H), jnp.float32),
)


def _edge_body(zd_ref, zs_ref, ea_ref, wf_ref, bf_ref, ws_ref, bs_ref, m_ref):
    zd = zd_ref[...]
    zs = zs_ref[...]
    ea_t = ea_ref[...]

    def gate(w_ref, b_ref):
        return (jnp.dot(zd, w_ref[0:H, :], preferred_element_type=jnp.float32)
                + jnp.dot(zs, w_ref[H:2 * H, :], preferred_element_type=jnp.float32)
                + lax.dot_general(ea_t, w_ref[2 * H:, :], (((0,), (0,)), ((), ())),
                                  preferred_element_type=jnp.float32)
                + b_ref[...])

    f = gate(wf_ref, bf_ref)
    s = gate(ws_ref, bs_ref)
    sig = 1.0 / (1.0 + jnp.exp(-f))
    sp = jnp.maximum(s, 0.0) + jnp.log(1.0 + jnp.exp(-jnp.abs(s)))
    m_ref[...] = sig * sp


def _make_edge(cnt, e0):
    blk0 = e0 // BE
    return pl.pallas_call(
        _edge_body,
        grid=(cnt // BE,),
        in_specs=[
            pl.BlockSpec((BE, H), lambda i: (i, 0)),
            pl.BlockSpec((BE, H), lambda i: (i, 0)),
            pl.BlockSpec((R, BE), lambda i: (0, blk0 + i)),
            pl.BlockSpec((2 * H + R, H), lambda i: (0, 0)),
            pl.BlockSpec((H,), lambda i: (0,)),
            pl.BlockSpec((2 * H + R, H), lambda i: (0, 0)),
            pl.BlockSpec((H,), lambda i: (0,)),
        ],
        out_specs=pl.BlockSpec((BE, H), lambda i: (i, 0)),
        out_shape=jax.ShapeDtypeStruct((cnt, H), jnp.float32),
    )


_edges = [_make_edge(cnt, e0) for e0, cnt, _ in EBLOCKS]


NAGG = 2 * len(EBLOCKS)


def _agg_sum(arefs):
    t = arefs[0][...]
    for a in arefs[1:]:
        t = t + a[...]
    return t


def _combine_body(h_ref, *rest):
    arefs, o_ref = rest[:NAGG], rest[NAGG]
    o_ref[...] = jnp.maximum(h_ref[...] + _agg_sum(arefs), 0.0)


_combine_call = pl.pallas_call(
    _combine_body,
    grid=(GN,),
    in_specs=[pl.BlockSpec((BN, H), lambda i: (i, 0)) for _ in range(1 + NAGG)],
    out_specs=pl.BlockSpec((BN, H), lambda i: (i, 0)),
    out_shape=jax.ShapeDtypeStruct((N, H), jnp.float32),
)


def _pool_body(h_ref, *rest):
    arefs = rest[:NAGG]
    b_ref, wl_ref, bl_ref, o_ref, sums, cnts = rest[NAGG:]
    i = pl.program_id(0)

    @pl.when(i == 0)
    def _():
        sums[...] = jnp.zeros_like(sums)
        cnts[...] = jnp.zeros_like(cnts)

    h3 = jnp.maximum(h_ref[...] + _agg_sum(arefs), 0.0)
    bb = b_ref[0, 0, :]
    onehot = (bb[:, None] == lax.broadcasted_iota(jnp.int32, (BN, G), 1)).astype(jnp.float32)
    sums[...] += lax.dot_general(onehot, h3, (((0,), (0,)), ((), ())),
                                 preferred_element_type=jnp.float32)
    cnts[...] += jnp.broadcast_to(jnp.sum(onehot, axis=0)[:, None], (G, H))

    @pl.when(i == GN - 1)
    def _():
        pooled = sums[...] / jnp.maximum(cnts[...], 1.0)
        o_ref[...] = jnp.dot(pooled, wl_ref[...],
                             preferred_element_type=jnp.float32) + bl_ref[...]


_pool_call = pl.pallas_call(
    _pool_body,
    grid=(GN,),
    in_specs=[pl.BlockSpec((BN, H), lambda i: (i, 0)) for _ in range(1 + NAGG)] + [
        pl.BlockSpec((1, 1, BN), lambda i: (i, 0, 0)),
        pl.BlockSpec((H, H), lambda i: (0, 0)),
        pl.BlockSpec((H,), lambda i: (0,)),
    ],
    out_specs=pl.BlockSpec((G, H), lambda i: (0, 0)),
    out_shape=jax.ShapeDtypeStruct((G, H), jnp.float32),
    scratch_shapes=[
        pltpu.VMEM((G, H), jnp.float32),
        pltpu.VMEM((G, H), jnp.float32),
    ],
)


def kernel(x, edge_index, edge_attr, batch, emb,
           Wf1, bf1, Ws1, bs1, Wf2, bf2, Ws2, bs2, Wf3, bf3, Ws3, bs3, Wl, bl):
    src = edge_index[0].astype(jnp.int32)
    dst = edge_index[1].astype(jnp.int32)
    eat = edge_attr.T
    dst3s = [dst[e0:e0 + cnt].reshape(NW, (cnt // NW) // ch, ch)
             for e0, cnt, ch in EBLOCKS]
    x3 = x.reshape(GN, 1, BN).astype(jnp.int32)
    b3 = batch.reshape(GN, 1, BN).astype(jnp.int32)
    zero = jnp.zeros((NPAD, H), jnp.float32)

    h = _emb_call(x3, emb)
    layers = [(Wf1, bf1, Ws1, bs1), (Wf2, bf2, Ws2, bs2), (Wf3, bf3, Ws3, bs3)]
    agg = None
    for li, (Wf, bf, Ws, bs) in enumerate(layers):
        if li > 0:
            h = _combine_call(h, *agg)
        zs = [g(h, src, dst) for g in _gathers]
        ms = [e(zsd[1], zsd[0], eat, Wf, bf, Ws, bs)
              for e, zsd in zip(_edges, zs)]
        agg = [a for s, m, d3 in zip(_scatters, ms, dst3s)
               for a in s(m, d3, zero)]

    return _pool_call(h, *agg, b3, Wl, bl)
